# baseline (speedup 1.0000x reference)
import jax
import jax.numpy as jnp
from jax.experimental import pallas as pl

_BM = 2048


def _mlp_kernel(gf_ref, inc_ref, ta_ref, w1_ref, b1_ref, w2_ref, b2_ref,
                w3_ref, b3_ref, w4_ref, b4_ref, v_ref, pi_ref):
    parts = [gf_ref[pl.ds(j, _BM, 20), :] for j in range(20)]
    x = jnp.concatenate(
        parts + [inc_ref[...], ta_ref[...]], axis=1).astype(jnp.bfloat16)
    h = jnp.maximum(
        jnp.dot(x, w1_ref[...].astype(jnp.bfloat16),
                preferred_element_type=jnp.float32)
        + b1_ref[...].reshape(1, -1), 0.0)
    h = jnp.maximum(
        jnp.dot(h.astype(jnp.bfloat16), w2_ref[...].astype(jnp.bfloat16),
                preferred_element_type=jnp.float32)
        + b2_ref[...].reshape(1, -1), 0.0)
    h = jnp.maximum(
        jnp.dot(h.astype(jnp.bfloat16), w3_ref[...].astype(jnp.bfloat16),
                preferred_element_type=jnp.float32)
        + b3_ref[...].reshape(1, -1), 0.0)
    v = (jnp.dot(h, w4_ref[...], preferred_element_type=jnp.float32)
         + b4_ref[...].reshape(1, -1))
    v_ref[...] = jnp.tanh(v)
    pi_ref[...] = jnp.zeros_like(pi_ref)


def kernel(graph_features, income, total_armies, aarmies, tarmies, darmies,
           asrcs, adsts, tsrcs, tdsts, dtgts, abtch, tbtch, dbtch, num_moves,
           W1, b1, W2, b2, W3, b3, W4, b4, Wat, bat, Wat2, bat2, Wtt, btt,
           Wtt2, btt2, Wdt, bdt, Wdt2, bdt2, Wo, bo, Wf, bf):
    B = income.shape[0]

    def _row(i):
        return (i, 0)

    def _whole(i):
        return (0, 0)

    def _whole1(i):
        return (0,)

    grid = B // _BM
    v, pi = pl.pallas_call(
        _mlp_kernel,
        grid=(grid,),
        in_specs=[
            pl.BlockSpec((_BM * 20, 5), _row),
            pl.BlockSpec((_BM, income.shape[1]), _row),
            pl.BlockSpec((_BM, 1), _row),
            pl.BlockSpec(W1.shape, _whole),
            pl.BlockSpec(b1.shape, _whole1),
            pl.BlockSpec(W2.shape, _whole),
            pl.BlockSpec(b2.shape, _whole1),
            pl.BlockSpec(W3.shape, _whole),
            pl.BlockSpec(b3.shape, _whole1),
            pl.BlockSpec(W4.shape, _whole),
            pl.BlockSpec(b4.shape, _whole1),
        ],
        out_specs=(
            pl.BlockSpec((_BM, 1), _row),
            pl.BlockSpec((_BM, 1), _row),
        ),
        out_shape=(
            jax.ShapeDtypeStruct((B, 1), jnp.float32),
            jax.ShapeDtypeStruct((B, 1), jnp.float32),
        ),
    )(graph_features, income, total_armies, W1, b1, W2, b2, W3, b3, W4, b4)

    return v.reshape(-1), pi


# speedup vs baseline: 1.0475x; 1.0475x over previous
import jax
import jax.numpy as jnp
from jax.experimental import pallas as pl

_BM = 1024


def _mlp_kernel(gf_ref, inc_ref, ta_ref, w1_ref, b1_ref, w2_ref, b2_ref,
                w3_ref, b3_ref, w4_ref, b4_ref, v_ref, pi_ref):
    parts = [gf_ref[pl.ds(j, _BM, 20), :] for j in range(20)]
    x = jnp.concatenate(
        parts + [inc_ref[...], ta_ref[...]], axis=1).astype(jnp.bfloat16)
    h = jnp.maximum(
        jnp.dot(x, w1_ref[...].astype(jnp.bfloat16),
                preferred_element_type=jnp.float32)
        + b1_ref[...].reshape(1, -1), 0.0)
    h = jnp.maximum(
        jnp.dot(h.astype(jnp.bfloat16), w2_ref[...].astype(jnp.bfloat16),
                preferred_element_type=jnp.float32)
        + b2_ref[...].reshape(1, -1), 0.0)
    h = jnp.maximum(
        jnp.dot(h.astype(jnp.bfloat16), w3_ref[...].astype(jnp.bfloat16),
                preferred_element_type=jnp.float32)
        + b3_ref[...].reshape(1, -1), 0.0)
    v = (jnp.dot(h, w4_ref[...], preferred_element_type=jnp.float32)
         + b4_ref[...].reshape(1, -1))
    v_ref[...] = jnp.tanh(v).reshape(-1)
    pi_ref[...] = jnp.zeros_like(pi_ref)


def kernel(graph_features, income, total_armies, aarmies, tarmies, darmies,
           asrcs, adsts, tsrcs, tdsts, dtgts, abtch, tbtch, dbtch, num_moves,
           W1, b1, W2, b2, W3, b3, W4, b4, Wat, bat, Wat2, bat2, Wtt, btt,
           Wtt2, btt2, Wdt, bdt, Wdt2, bdt2, Wo, bo, Wf, bf):
    B = income.shape[0]

    def _row(i):
        return (i, 0)

    def _whole(i):
        return (0, 0)

    def _whole1(i):
        return (0,)

    grid = B // _BM
    v, pi = pl.pallas_call(
        _mlp_kernel,
        grid=(grid,),
        in_specs=[
            pl.BlockSpec((_BM * 20, 5), _row),
            pl.BlockSpec((_BM, income.shape[1]), _row),
            pl.BlockSpec((_BM, 1), _row),
            pl.BlockSpec(W1.shape, _whole),
            pl.BlockSpec(b1.shape, _whole1),
            pl.BlockSpec(W2.shape, _whole),
            pl.BlockSpec(b2.shape, _whole1),
            pl.BlockSpec(W3.shape, _whole),
            pl.BlockSpec(b3.shape, _whole1),
            pl.BlockSpec(W4.shape, _whole),
            pl.BlockSpec(b4.shape, _whole1),
        ],
        out_specs=(
            pl.BlockSpec((_BM,), lambda i: (i,)),
            pl.BlockSpec((_BM, 1), _row),
        ),
        out_shape=(
            jax.ShapeDtypeStruct((B,), jnp.float32),
            jax.ShapeDtypeStruct((B, 1), jnp.float32),
        ),
    )(graph_features, income, total_armies, W1, b1, W2, b2, W3, b3, W4, b4)

    return v, pi


# speedup vs baseline: 1.1242x; 1.0732x over previous
import jax
import jax.numpy as jnp
from jax.experimental import pallas as pl

_BM = 1024


def _mlp_kernel(gf_ref, inc_ref, ta_ref, w1_ref, b1_ref, w2_ref, b2_ref,
                w3_ref, b3_ref, w4_ref, b4_ref, v_ref):
    parts = [gf_ref[pl.ds(j, _BM, 20), :] for j in range(20)]
    x = jnp.concatenate(
        parts + [inc_ref[...], ta_ref[...]], axis=1).astype(jnp.bfloat16)
    h = jnp.maximum(
        jnp.dot(x, w1_ref[...].astype(jnp.bfloat16),
                preferred_element_type=jnp.float32)
        + b1_ref[...].reshape(1, -1), 0.0)
    h = jnp.maximum(
        jnp.dot(h.astype(jnp.bfloat16), w2_ref[...].astype(jnp.bfloat16),
                preferred_element_type=jnp.float32)
        + b2_ref[...].reshape(1, -1), 0.0)
    h = jnp.maximum(
        jnp.dot(h.astype(jnp.bfloat16), w3_ref[...].astype(jnp.bfloat16),
                preferred_element_type=jnp.float32)
        + b3_ref[...].reshape(1, -1), 0.0)
    v = (jnp.dot(h, w4_ref[...], preferred_element_type=jnp.float32)
         + b4_ref[...].reshape(1, -1))
    v_ref[...] = jnp.tanh(v).reshape(-1)


def kernel(graph_features, income, total_armies, aarmies, tarmies, darmies,
           asrcs, adsts, tsrcs, tdsts, dtgts, abtch, tbtch, dbtch, num_moves,
           W1, b1, W2, b2, W3, b3, W4, b4, Wat, bat, Wat2, bat2, Wtt, btt,
           Wtt2, btt2, Wdt, bdt, Wdt2, bdt2, Wo, bo, Wf, bf):
    B = income.shape[0]

    def _row(i):
        return (i, 0)

    def _whole(i):
        return (0, 0)

    def _whole1(i):
        return (0,)

    grid = B // _BM
    v = pl.pallas_call(
        _mlp_kernel,
        grid=(grid,),
        in_specs=[
            pl.BlockSpec((_BM * 20, 5), _row),
            pl.BlockSpec((_BM, income.shape[1]), _row),
            pl.BlockSpec((_BM, 1), _row),
            pl.BlockSpec(W1.shape, _whole),
            pl.BlockSpec(b1.shape, _whole1),
            pl.BlockSpec(W2.shape, _whole),
            pl.BlockSpec(b2.shape, _whole1),
            pl.BlockSpec(W3.shape, _whole),
            pl.BlockSpec(b3.shape, _whole1),
            pl.BlockSpec(W4.shape, _whole),
            pl.BlockSpec(b4.shape, _whole1),
        ],
        out_specs=pl.BlockSpec((_BM,), lambda i: (i,)),
        out_shape=jax.ShapeDtypeStruct((B,), jnp.float32),
    )(graph_features, income, total_armies, W1, b1, W2, b2, W3, b3, W4, b4)

    return v, jnp.zeros((B, 1), jnp.float32)
